# Initial kernel scaffold; baseline (speedup 1.0000x reference)
import jax
import jax.numpy as jnp
from jax import lax
from jax.experimental import pallas as pl
from jax.experimental.pallas import tpu as pltpu
from jax.experimental.pallas import tpu_sc as plsc

NN = 10000
NE = 320000
PK = 16
MR = NE // PK
NR = NN // PK
BM = 2000
NC, NS = 2, 16
NW = NC * NS
EPW = NE // NW
C = 125
K = EPW // C
GSZ = 16
NG = K // GSZ
NPT = NN // NS

_f32 = jnp.float32


def _bd(w):
    return jnp.kron(jnp.eye(PK, dtype=w.dtype), w)


def _bt(b):
    return jnp.tile(b, PK)[None, :]


def _node_enc_body(x_ref, w1, b1, w2, b2, o_ref):
    z = jnp.dot(x_ref[...], w1[...], preferred_element_type=_f32) + b1[...]
    z = jnp.maximum(z, 0.0)
    h = jnp.dot(z, w2[...], preferred_element_type=_f32) + b2[...]
    o_ref[...] = jnp.maximum(h, 0.0)


def _edge_enc_body(ea_ref, w1, b1, w2, b2, o_ref):
    z = jnp.dot(ea_ref[...], w1[...], preferred_element_type=_f32) + b1[...]
    z = jnp.maximum(z, 0.0)
    z = jnp.dot(z, w2[...], preferred_element_type=_f32) + b2[...]
    o_ref[...] = jnp.maximum(z, 0.0)


def _rel_body(gd_ref, gs_ref, e_ref, w1d, w1s, w1e, b1, w2, b2, w3, b3, o_ref):
    z = (jnp.dot(gd_ref[...], w1d[...], preferred_element_type=_f32)
         + jnp.dot(gs_ref[...], w1s[...], preferred_element_type=_f32)
         + jnp.dot(e_ref[...], w1e[...], preferred_element_type=_f32)
         + b1[...])
    z = jnp.maximum(z, 0.0)
    z = jnp.maximum(jnp.dot(z, w2[...], preferred_element_type=_f32) + b2[...], 0.0)
    o_ref[...] = jnp.dot(z, w3[...], preferred_element_type=_f32) + b3[...]


def _obj_body(h_ref, p0_ref, p1_ref, w1h, w1a, b1, w2, b2, w3, b3, o_ref):
    h = h_ref[...]
    aggr = p0_ref[...] + p1_ref[...]
    z = (jnp.dot(h, w1h[...], preferred_element_type=_f32)
         + jnp.dot(aggr, w1a[...], preferred_element_type=_f32)
         + b1[...])
    z = jnp.maximum(z, 0.0)
    z = jnp.maximum(jnp.dot(z, w2[...], preferred_element_type=_f32) + b2[...], 0.0)
    delta = jnp.dot(z, w3[...], preferred_element_type=_f32) + b3[...]
    o_ref[...] = 0.5 * h + 0.5 * jnp.maximum(delta, 0.0)


def _final_body(e0, e1, e2, e3, e4, e5, q0, q1, q2, q3, q4, q5,
                b1, w2, b2, w3, b3, o_ref):
    es = (e0, e1, e2, e3, e4, e5)
    qs = (q0, q1, q2, q3, q4, q5)
    z = b1[...]
    for e, q in zip(es, qs):
        z = z + jnp.dot(e[...], q[...], preferred_element_type=_f32)
    z = jnp.maximum(z, 0.0)
    z = jnp.maximum(jnp.dot(z, w2[...], preferred_element_type=_f32) + b2[...], 0.0)
    z = jnp.dot(z, w3[...], preferred_element_type=_f32) + b3[...]
    o_ref[...] = 1.0 / (1.0 + jnp.exp(-z))


def _full(shape):
    return pl.BlockSpec(shape, lambda i: tuple(0 for _ in shape))


def _rows(width):
    return pl.BlockSpec((BM, width), lambda i: (i, 0))


def _sc_gather_body(h_hbm, dst3, src3, gd_hbm, gs_hbm, idx_v, rows_v, sem):
    c = lax.axis_index("c")
    s = lax.axis_index("s")
    w = c * NS + s
    base = w * EPW

    def one_table(tab3, out_hbm):
        pltpu.sync_copy(tab3.at[w], idx_v)

        @pl.loop(0, NG)
        def _(g):
            cps = []
            for j in range(GSZ):
                k = g * GSZ + j
                cps.append(pltpu.async_copy(
                    h_hbm.at[idx_v.at[k]], rows_v.at[pl.ds(k * C, C)], sem))
            for cp in cps:
                cp.wait()

        pltpu.sync_copy(rows_v, out_hbm.at[pl.ds(base, EPW)])

    one_table(dst3, gd_hbm)
    one_table(src3, gs_hbm)


def _sc_scatter_body(e_hbm, dst3, zeros_hbm, out_hbm, aggr_sh, idx_v, e_v, sem):
    c = lax.axis_index("c")
    s = lax.axis_index("s")
    w = c * NS + s
    base = w * EPW
    nb = s * NPT

    pltpu.sync_copy(zeros_hbm.at[pl.ds(nb, NPT)], aggr_sh.at[pl.ds(nb, NPT)])
    pltpu.sync_copy(dst3.at[w], idx_v)
    pltpu.sync_copy(e_hbm.at[pl.ds(base, EPW)], e_v)
    plsc.subcore_barrier()

    @pl.loop(0, NG)
    def _(g):
        cps = []
        for j in range(GSZ):
            k = g * GSZ + j
            cps.append(pltpu.async_copy(
                e_v.at[pl.ds(k * C, C)], aggr_sh.at[idx_v.at[k]], sem, add=True))
        for cp in cps:
            cp.wait()

    plsc.subcore_barrier()
    pltpu.sync_copy(aggr_sh.at[pl.ds(nb, NPT)], out_hbm.at[c, pl.ds(nb, NPT)])


_sc_mesh = plsc.VectorSubcoreMesh(core_axis_name="c", subcore_axis_name="s")

_sc_gather = pl.kernel(
    _sc_gather_body,
    out_type=(jax.ShapeDtypeStruct((NE, 8), _f32),
              jax.ShapeDtypeStruct((NE, 8), _f32)),
    mesh=_sc_mesh,
    scratch_types=[
        pltpu.VMEM((K, C), jnp.int32),
        pltpu.VMEM((EPW, 8), _f32),
        pltpu.SemaphoreType.DMA,
    ],
)

_sc_scatter = pl.kernel(
    _sc_scatter_body,
    out_type=jax.ShapeDtypeStruct((NC, NN, 8), _f32),
    mesh=_sc_mesh,
    scratch_types=[
        pltpu.VMEM_SHARED((NN, 8), _f32),
        pltpu.VMEM((K, C), jnp.int32),
        pltpu.VMEM((EPW, 8), _f32),
        pltpu.SemaphoreType.DMA,
    ],
)


@jax.jit
def _run(x, edge_attr, edge_index, params):
    src3 = edge_index[0].astype(jnp.int32).reshape(NW, K, C)
    dst3 = edge_index[1].astype(jnp.int32).reshape(NW, K, C)

    (wn1, bn1), (wn2, bn2) = params['node_encoder']
    (we1, be1), (we2, be2) = params['edge_encoder']
    layers = []
    for lp in params['layers']:
        (rw1, rb1), (rw2, rb2), (rw3, rb3) = lp['relational']
        (ow1, ob1), (ow2, ob2), (ow3, ob3) = lp['object']
        layers.append(dict(
            w1d=_bd(rw1[0:8]), w1s=_bd(rw1[8:16]), w1e=_bd(rw1[16:24]),
            b1=_bt(rb1), w2=_bd(rw2), b2=_bt(rb2), w3=_bd(rw3), b3=_bt(rb3),
            ow1h=_bd(ow1[0:8]), ow1a=_bd(ow1[8:16]), ob1=_bt(ob1),
            ow2=_bd(ow2), ob2=_bt(ob2), ow3=_bd(ow3), ob3=_bt(ob3),
        ))
    (fw1, fb1), (fw2, fb2), (fw3, fb3) = params['W']
    fq = [_bd(fw1[8 * l:8 * (l + 1)]) for l in range(6)]

    h = pl.pallas_call(
        _node_enc_body,
        out_shape=jax.ShapeDtypeStruct((NN, 8), _f32),
    )(x, wn1, bn1[None, :], wn2, bn2[None, :])

    ea2 = edge_attr.reshape(MR, 256)
    e2 = pl.pallas_call(
        _edge_enc_body,
        grid=(MR // BM,),
        in_specs=[_rows(256), _full((256, 256)), _full((1, 256)),
                  _full((256, 128)), _full((1, 128))],
        out_specs=_rows(128),
        out_shape=jax.ShapeDtypeStruct((MR, 128), _f32),
    )(ea2, _bd(we1), _bt(be1), _bd(we2), _bt(be2))

    zeros = jnp.zeros((NN, 8), _f32)
    e_list = []
    for l in range(6):
        lw = layers[l]
        gd, gs = _sc_gather(h, dst3, src3)
        e2 = pl.pallas_call(
            _rel_body,
            grid=(MR // BM,),
            in_specs=[_rows(128), _rows(128), _rows(128),
                      _full((128, 256)), _full((128, 256)), _full((128, 256)),
                      _full((1, 256)), _full((256, 256)), _full((1, 256)),
                      _full((256, 128)), _full((1, 128))],
            out_specs=_rows(128),
            out_shape=jax.ShapeDtypeStruct((MR, 128), _f32),
        )(gd.reshape(MR, 128), gs.reshape(MR, 128), e2,
          lw['w1d'], lw['w1s'], lw['w1e'], lw['b1'],
          lw['w2'], lw['b2'], lw['w3'], lw['b3'])
        e_list.append(e2)
        p = _sc_scatter(e2.reshape(NE, 8), dst3, zeros)
        h2 = pl.pallas_call(
            _obj_body,
            out_shape=jax.ShapeDtypeStruct((NR, 128), _f32),
        )(h.reshape(NR, 128), p[0].reshape(NR, 128), p[1].reshape(NR, 128),
          lw['ow1h'], lw['ow1a'], lw['ob1'], lw['ow2'], lw['ob2'],
          lw['ow3'], lw['ob3'])
        h = h2.reshape(NN, 8)

    wout = pl.pallas_call(
        _final_body,
        grid=(MR // BM,),
        in_specs=([_rows(128)] * 6 + [_full((128, 256))] * 6
                  + [_full((1, 256)), _full((256, 256)), _full((1, 256)),
                     _full((256, 16)), _full((1, 16))]),
        out_specs=_rows(16),
        out_shape=jax.ShapeDtypeStruct((MR, 16), _f32),
    )(*e_list, *fq, _bt(fb1), _bd(fw2), _bt(fb2), _bd(fw3), _bt(fb3))
    return wout.reshape(NE)


def kernel(x, edge_attr, edge_index, params):
    return _run(x, edge_attr, edge_index, params)


# speedup vs baseline: 14.8122x; 14.8122x over previous
import jax
import jax.numpy as jnp
from jax import lax
from jax.experimental import pallas as pl
from jax.experimental.pallas import tpu as pltpu
from jax.experimental.pallas import tpu_sc as plsc

NN = 10000
NE = 320000
PK = 16
MR = NE // PK
NR = NN // PK
BM = 2000
NC, NS = 2, 16
NW = NC * NS
EPW = NE // NW
C = 125
K = EPW // C
GSZ = 16
NG = K // GSZ
NPT = NN // NS

_f32 = jnp.float32


def _bd(w):
    return jnp.kron(jnp.eye(PK, dtype=w.dtype), w)


def _bt(b):
    return jnp.tile(b, PK)[None, :]


def _node_enc_body(x_ref, w1, b1, w2, b2, o_ref):
    z = jnp.dot(x_ref[...], w1[...], preferred_element_type=_f32) + b1[...]
    z = jnp.maximum(z, 0.0)
    h = jnp.dot(z, w2[...], preferred_element_type=_f32) + b2[...]
    o_ref[...] = jnp.maximum(h, 0.0)


def _edge_enc_body(ea_ref, w1, b1, w2, b2, o_ref):
    z = jnp.dot(ea_ref[...], w1[...], preferred_element_type=_f32) + b1[...]
    z = jnp.maximum(z, 0.0)
    z = jnp.dot(z, w2[...], preferred_element_type=_f32) + b2[...]
    o_ref[...] = jnp.maximum(z, 0.0)


def _rel_body(gd_ref, gs_ref, e_ref, w1d, w1s, w1e, b1, w2, b2, w3, b3, o_ref):
    z = (jnp.dot(gd_ref[...], w1d[...], preferred_element_type=_f32)
         + jnp.dot(gs_ref[...], w1s[...], preferred_element_type=_f32)
         + jnp.dot(e_ref[...], w1e[...], preferred_element_type=_f32)
         + b1[...])
    z = jnp.maximum(z, 0.0)
    z = jnp.maximum(jnp.dot(z, w2[...], preferred_element_type=_f32) + b2[...], 0.0)
    o_ref[...] = jnp.dot(z, w3[...], preferred_element_type=_f32) + b3[...]


def _obj_body(h_ref, p0_ref, p1_ref, w1h, w1a, b1, w2, b2, w3, b3, o_ref):
    h = h_ref[...]
    aggr = p0_ref[...] + p1_ref[...]
    z = (jnp.dot(h, w1h[...], preferred_element_type=_f32)
         + jnp.dot(aggr, w1a[...], preferred_element_type=_f32)
         + b1[...])
    z = jnp.maximum(z, 0.0)
    z = jnp.maximum(jnp.dot(z, w2[...], preferred_element_type=_f32) + b2[...], 0.0)
    delta = jnp.dot(z, w3[...], preferred_element_type=_f32) + b3[...]
    o_ref[...] = 0.5 * h + 0.5 * jnp.maximum(delta, 0.0)


def _final_body(e0, e1, e2, e3, e4, e5, q0, q1, q2, q3, q4, q5,
                b1, w2, b2, w3, b3, o_ref):
    es = (e0, e1, e2, e3, e4, e5)
    qs = (q0, q1, q2, q3, q4, q5)
    z = b1[...]
    for e, q in zip(es, qs):
        z = z + jnp.dot(e[...], q[...], preferred_element_type=_f32)
    z = jnp.maximum(z, 0.0)
    z = jnp.maximum(jnp.dot(z, w2[...], preferred_element_type=_f32) + b2[...], 0.0)
    z = jnp.dot(z, w3[...], preferred_element_type=_f32) + b3[...]
    o_ref[...] = 1.0 / (1.0 + jnp.exp(-z))


def _full(shape):
    return pl.BlockSpec(shape, lambda i: tuple(0 for _ in shape))


def _rows(width):
    return pl.BlockSpec((BM, width), lambda i: (i, 0))


def _sc_gather_body(h_hbm, dst3, src3, gd_hbm, gs_hbm, idx_v, rows_v, sem):
    c = lax.axis_index("c")
    s = lax.axis_index("s")
    w = c * NS + s
    base = w * EPW

    def one_table(tab3, out_hbm):
        pltpu.sync_copy(tab3.at[w], idx_v)

        @pl.loop(0, NG)
        def _(g):
            cps = []
            for j in range(GSZ):
                k = g * GSZ + j
                cps.append(pltpu.async_copy(
                    h_hbm.at[idx_v.at[k]], rows_v.at[pl.ds(k * C, C)], sem))
            for cp in cps:
                cp.wait()

        pltpu.sync_copy(rows_v, out_hbm.at[pl.ds(base, EPW)])

    one_table(dst3, gd_hbm)
    one_table(src3, gs_hbm)


def _sc_scatter_body(e_hbm, dst3, zeros_hbm, out_hbm, aggr_sh, idx_v, e_v, sem):
    c = lax.axis_index("c")
    s = lax.axis_index("s")
    w = c * NS + s
    base = w * EPW
    nb = s * NPT

    pltpu.sync_copy(zeros_hbm.at[pl.ds(nb, NPT)], aggr_sh.at[pl.ds(nb, NPT)])
    pltpu.sync_copy(dst3.at[w], idx_v)
    pltpu.sync_copy(e_hbm.at[pl.ds(base, EPW)], e_v)
    plsc.subcore_barrier()

    @pl.loop(0, NG)
    def _(g):
        cps = []
        for j in range(GSZ):
            k = g * GSZ + j
            cps.append(pltpu.async_copy(
                e_v.at[pl.ds(k * C, C)], aggr_sh.at[idx_v.at[k]], sem, add=True))
        for cp in cps:
            cp.wait()

    plsc.subcore_barrier()
    pltpu.sync_copy(aggr_sh.at[pl.ds(nb, NPT)], out_hbm.at[c, pl.ds(nb, NPT)])


import functools


@functools.lru_cache(maxsize=None)
def _sc_kernels():
    mesh = plsc.VectorSubcoreMesh(
        core_axis_name="c", subcore_axis_name="s",
        num_cores=NC, num_subcores=NS)
    params = pltpu.CompilerParams(use_tc_tiling_on_sc=False)
    gather = pl.kernel(
        _sc_gather_body,
        out_type=(jax.ShapeDtypeStruct((NE, 8), _f32),
                  jax.ShapeDtypeStruct((NE, 8), _f32)),
        mesh=mesh,
        compiler_params=params,
        scratch_types=[
            pltpu.VMEM((K, C), jnp.int32),
            pltpu.VMEM((EPW, 8), _f32),
            pltpu.SemaphoreType.DMA,
        ],
    )
    scatter = pl.kernel(
        _sc_scatter_body,
        out_type=jax.ShapeDtypeStruct((NC, NN, 8), _f32),
        mesh=mesh,
        compiler_params=params,
        scratch_types=[
            pltpu.VMEM_SHARED((NN, 8), _f32),
            pltpu.VMEM((K, C), jnp.int32),
            pltpu.VMEM((EPW, 8), _f32),
            pltpu.SemaphoreType.DMA,
        ],
    )
    return gather, scatter


@jax.jit
def _run(x, edge_attr, edge_index, params):
    src3 = edge_index[0].astype(jnp.int32).reshape(NW, K, C)
    dst3 = edge_index[1].astype(jnp.int32).reshape(NW, K, C)

    (wn1, bn1), (wn2, bn2) = params['node_encoder']
    (we1, be1), (we2, be2) = params['edge_encoder']
    layers = []
    for lp in params['layers']:
        (rw1, rb1), (rw2, rb2), (rw3, rb3) = lp['relational']
        (ow1, ob1), (ow2, ob2), (ow3, ob3) = lp['object']
        layers.append(dict(
            w1d=_bd(rw1[0:8]), w1s=_bd(rw1[8:16]), w1e=_bd(rw1[16:24]),
            b1=_bt(rb1), w2=_bd(rw2), b2=_bt(rb2), w3=_bd(rw3), b3=_bt(rb3),
            ow1h=_bd(ow1[0:8]), ow1a=_bd(ow1[8:16]), ob1=_bt(ob1),
            ow2=_bd(ow2), ob2=_bt(ob2), ow3=_bd(ow3), ob3=_bt(ob3),
        ))
    (fw1, fb1), (fw2, fb2), (fw3, fb3) = params['W']
    fq = [_bd(fw1[8 * l:8 * (l + 1)]) for l in range(6)]

    h = pl.pallas_call(
        _node_enc_body,
        out_shape=jax.ShapeDtypeStruct((NN, 8), _f32),
    )(x, wn1, bn1[None, :], wn2, bn2[None, :])

    ea2 = edge_attr.reshape(MR, 256)
    e2 = pl.pallas_call(
        _edge_enc_body,
        grid=(MR // BM,),
        in_specs=[_rows(256), _full((256, 256)), _full((1, 256)),
                  _full((256, 128)), _full((1, 128))],
        out_specs=_rows(128),
        out_shape=jax.ShapeDtypeStruct((MR, 128), _f32),
    )(ea2, _bd(we1), _bt(be1), _bd(we2), _bt(be2))

    sc_gather, sc_scatter = _sc_kernels()
    zeros = jnp.zeros((NN, 8), _f32)
    e_list = []
    for l in range(6):
        lw = layers[l]
        gd, gs = sc_gather(h, dst3, src3)
        e2 = pl.pallas_call(
            _rel_body,
            grid=(MR // BM,),
            in_specs=[_rows(128), _rows(128), _rows(128),
                      _full((128, 256)), _full((128, 256)), _full((128, 256)),
                      _full((1, 256)), _full((256, 256)), _full((1, 256)),
                      _full((256, 128)), _full((1, 128))],
            out_specs=_rows(128),
            out_shape=jax.ShapeDtypeStruct((MR, 128), _f32),
        )(gd.reshape(MR, 128), gs.reshape(MR, 128), e2,
          lw['w1d'], lw['w1s'], lw['w1e'], lw['b1'],
          lw['w2'], lw['b2'], lw['w3'], lw['b3'])
        e_list.append(e2)
        p = sc_scatter(e2.reshape(NE, 8), dst3, zeros)
        h2 = pl.pallas_call(
            _obj_body,
            out_shape=jax.ShapeDtypeStruct((NR, 128), _f32),
        )(h.reshape(NR, 128), p[0].reshape(NR, 128), p[1].reshape(NR, 128),
          lw['ow1h'], lw['ow1a'], lw['ob1'], lw['ow2'], lw['ob2'],
          lw['ow3'], lw['ob3'])
        h = h2.reshape(NN, 8)

    wout = pl.pallas_call(
        _final_body,
        grid=(MR // BM,),
        in_specs=([_rows(128)] * 6 + [_full((128, 256))] * 6
                  + [_full((1, 256)), _full((256, 256)), _full((1, 256)),
                     _full((256, 16)), _full((1, 16))]),
        out_specs=_rows(16),
        out_shape=jax.ShapeDtypeStruct((MR, 16), _f32),
    )(*e_list, *fq, _bt(fb1), _bd(fw2), _bt(fb2), _bd(fw3), _bt(fb3))
    return wout.reshape(NE)


def kernel(x, edge_attr, edge_index, params):
    return _run(x, edge_attr, edge_index, params)


# speedup vs baseline: 15.3095x; 1.0336x over previous
import jax
import jax.numpy as jnp
from jax import lax
from jax.experimental import pallas as pl
from jax.experimental.pallas import tpu as pltpu
from jax.experimental.pallas import tpu_sc as plsc

NN = 10000
NE = 320000
PK = 16
MR = NE // PK
NR = NN // PK
BM = 2000
NC, NS = 2, 16
NW = NC * NS
EPW = NE // NW
C = 125
K = EPW // C
GSZ = 16
NG = K // GSZ
NPT = NN // NS

_f32 = jnp.float32


def _bd(w):
    return jnp.kron(jnp.eye(PK, dtype=w.dtype), w)


def _bt(b):
    return jnp.tile(b, PK)[None, :]


def _node_enc_body(x_ref, w1, b1, w2, b2, o_ref):
    z = jnp.dot(x_ref[...], w1[...], preferred_element_type=_f32) + b1[...]
    z = jnp.maximum(z, 0.0)
    h = jnp.dot(z, w2[...], preferred_element_type=_f32) + b2[...]
    o_ref[...] = jnp.maximum(h, 0.0)


def _edge_enc_body(ea_ref, w1, b1, w2, b2, o_ref):
    z = jnp.dot(ea_ref[...], w1[...], preferred_element_type=_f32) + b1[...]
    z = jnp.maximum(z, 0.0)
    z = jnp.dot(z, w2[...], preferred_element_type=_f32) + b2[...]
    o_ref[...] = jnp.maximum(z, 0.0)


def _rel_body(gd_ref, gs_ref, e_ref, w1d, w1s, w1e, b1, w2, b2, w3, b3, o_ref):
    z = (jnp.dot(gd_ref[...], w1d[...], preferred_element_type=_f32)
         + jnp.dot(gs_ref[...], w1s[...], preferred_element_type=_f32)
         + jnp.dot(e_ref[...], w1e[...], preferred_element_type=_f32)
         + b1[...])
    z = jnp.maximum(z, 0.0)
    z = jnp.maximum(jnp.dot(z, w2[...], preferred_element_type=_f32) + b2[...], 0.0)
    o_ref[...] = jnp.dot(z, w3[...], preferred_element_type=_f32) + b3[...]


def _obj_body(h_ref, p0_ref, p1_ref, w1h, w1a, b1, w2, b2, w3, b3, o_ref):
    h = h_ref[...]
    aggr = p0_ref[...] + p1_ref[...]
    z = (jnp.dot(h, w1h[...], preferred_element_type=_f32)
         + jnp.dot(aggr, w1a[...], preferred_element_type=_f32)
         + b1[...])
    z = jnp.maximum(z, 0.0)
    z = jnp.maximum(jnp.dot(z, w2[...], preferred_element_type=_f32) + b2[...], 0.0)
    delta = jnp.dot(z, w3[...], preferred_element_type=_f32) + b3[...]
    o_ref[...] = 0.5 * h + 0.5 * jnp.maximum(delta, 0.0)


def _final_body(e0, e1, e2, e3, e4, e5, q0, q1, q2, q3, q4, q5,
                b1, w2, b2, w3, b3, o_ref):
    es = (e0, e1, e2, e3, e4, e5)
    qs = (q0, q1, q2, q3, q4, q5)
    z = b1[...]
    for e, q in zip(es, qs):
        z = z + jnp.dot(e[...], q[...], preferred_element_type=_f32)
    z = jnp.maximum(z, 0.0)
    z = jnp.maximum(jnp.dot(z, w2[...], preferred_element_type=_f32) + b2[...], 0.0)
    z = jnp.dot(z, w3[...], preferred_element_type=_f32) + b3[...]
    o_ref[...] = 1.0 / (1.0 + jnp.exp(-z))


def _full(shape):
    return pl.BlockSpec(shape, lambda i: tuple(0 for _ in shape))


def _rows(width):
    return pl.BlockSpec((BM, width), lambda i: (i, 0))


def _sc_gather_body(h_hbm, dst2, src2, gd_hbm, gs_hbm, idx_v, rows_v, sem):
    c = lax.axis_index("c")
    s = lax.axis_index("s")
    w = c * NS + s
    base = w * EPW

    def one_table(tab2, out_hbm):
        pltpu.sync_copy(tab2.at[w], idx_v)
        pltpu.async_copy(h_hbm.at[idx_v], rows_v, sem).wait()
        pltpu.sync_copy(rows_v, out_hbm.at[pl.ds(base, EPW)])

    one_table(dst2, gd_hbm)
    one_table(src2, gs_hbm)


def _sc_scatter_body(e_hbm, dst2, zeros_hbm, out_hbm, aggr_sh, idx_v, e_v, sem):
    c = lax.axis_index("c")
    s = lax.axis_index("s")
    w = c * NS + s
    base = w * EPW
    nb = s * NPT

    pltpu.sync_copy(zeros_hbm.at[pl.ds(nb, NPT)], aggr_sh.at[pl.ds(nb, NPT)])
    pltpu.sync_copy(dst2.at[w], idx_v)
    pltpu.sync_copy(e_hbm.at[pl.ds(base, EPW)], e_v)
    plsc.subcore_barrier()

    pltpu.async_copy(e_v, aggr_sh.at[idx_v], sem, add=True).wait()

    plsc.subcore_barrier()
    pltpu.sync_copy(aggr_sh.at[pl.ds(nb, NPT)], out_hbm.at[c, pl.ds(nb, NPT)])


import functools


@functools.lru_cache(maxsize=None)
def _sc_kernels():
    mesh = plsc.VectorSubcoreMesh(
        core_axis_name="c", subcore_axis_name="s",
        num_cores=NC, num_subcores=NS)
    params = pltpu.CompilerParams(use_tc_tiling_on_sc=False)
    gather = pl.kernel(
        _sc_gather_body,
        out_type=(jax.ShapeDtypeStruct((NE, 8), _f32),
                  jax.ShapeDtypeStruct((NE, 8), _f32)),
        mesh=mesh,
        compiler_params=params,
        scratch_types=[
            pltpu.VMEM((EPW,), jnp.int32),
            pltpu.VMEM((EPW, 8), _f32),
            pltpu.SemaphoreType.DMA,
        ],
    )
    scatter = pl.kernel(
        _sc_scatter_body,
        out_type=jax.ShapeDtypeStruct((NC, NN, 8), _f32),
        mesh=mesh,
        compiler_params=params,
        scratch_types=[
            pltpu.VMEM_SHARED((NN, 8), _f32),
            pltpu.VMEM((EPW,), jnp.int32),
            pltpu.VMEM((EPW, 8), _f32),
            pltpu.SemaphoreType.DMA,
        ],
    )
    return gather, scatter


@jax.jit
def _run(x, edge_attr, edge_index, params):
    src2 = edge_index[0].astype(jnp.int32).reshape(NW, EPW)
    dst2 = edge_index[1].astype(jnp.int32).reshape(NW, EPW)

    (wn1, bn1), (wn2, bn2) = params['node_encoder']
    (we1, be1), (we2, be2) = params['edge_encoder']
    layers = []
    for lp in params['layers']:
        (rw1, rb1), (rw2, rb2), (rw3, rb3) = lp['relational']
        (ow1, ob1), (ow2, ob2), (ow3, ob3) = lp['object']
        layers.append(dict(
            w1d=_bd(rw1[0:8]), w1s=_bd(rw1[8:16]), w1e=_bd(rw1[16:24]),
            b1=_bt(rb1), w2=_bd(rw2), b2=_bt(rb2), w3=_bd(rw3), b3=_bt(rb3),
            ow1h=_bd(ow1[0:8]), ow1a=_bd(ow1[8:16]), ob1=_bt(ob1),
            ow2=_bd(ow2), ob2=_bt(ob2), ow3=_bd(ow3), ob3=_bt(ob3),
        ))
    (fw1, fb1), (fw2, fb2), (fw3, fb3) = params['W']
    fq = [_bd(fw1[8 * l:8 * (l + 1)]) for l in range(6)]

    h = pl.pallas_call(
        _node_enc_body,
        out_shape=jax.ShapeDtypeStruct((NN, 8), _f32),
    )(x, wn1, bn1[None, :], wn2, bn2[None, :])

    ea2 = edge_attr.reshape(MR, 256)
    e2 = pl.pallas_call(
        _edge_enc_body,
        grid=(MR // BM,),
        in_specs=[_rows(256), _full((256, 256)), _full((1, 256)),
                  _full((256, 128)), _full((1, 128))],
        out_specs=_rows(128),
        out_shape=jax.ShapeDtypeStruct((MR, 128), _f32),
    )(ea2, _bd(we1), _bt(be1), _bd(we2), _bt(be2))

    sc_gather, sc_scatter = _sc_kernels()
    zeros = jnp.zeros((NN, 8), _f32)
    e_list = []
    for l in range(6):
        lw = layers[l]
        gd, gs = sc_gather(h, dst2, src2)
        e2 = pl.pallas_call(
            _rel_body,
            grid=(MR // BM,),
            in_specs=[_rows(128), _rows(128), _rows(128),
                      _full((128, 256)), _full((128, 256)), _full((128, 256)),
                      _full((1, 256)), _full((256, 256)), _full((1, 256)),
                      _full((256, 128)), _full((1, 128))],
            out_specs=_rows(128),
            out_shape=jax.ShapeDtypeStruct((MR, 128), _f32),
        )(gd.reshape(MR, 128), gs.reshape(MR, 128), e2,
          lw['w1d'], lw['w1s'], lw['w1e'], lw['b1'],
          lw['w2'], lw['b2'], lw['w3'], lw['b3'])
        e_list.append(e2)
        p = sc_scatter(e2.reshape(NE, 8), dst2, zeros)
        h2 = pl.pallas_call(
            _obj_body,
            out_shape=jax.ShapeDtypeStruct((NR, 128), _f32),
        )(h.reshape(NR, 128), p[0].reshape(NR, 128), p[1].reshape(NR, 128),
          lw['ow1h'], lw['ow1a'], lw['ob1'], lw['ow2'], lw['ob2'],
          lw['ow3'], lw['ob3'])
        h = h2.reshape(NN, 8)

    wout = pl.pallas_call(
        _final_body,
        grid=(MR // BM,),
        in_specs=([_rows(128)] * 6 + [_full((128, 256))] * 6
                  + [_full((1, 256)), _full((256, 256)), _full((1, 256)),
                     _full((256, 16)), _full((1, 16))]),
        out_specs=_rows(16),
        out_shape=jax.ShapeDtypeStruct((MR, 16), _f32),
    )(*e_list, *fq, _bt(fb1), _bd(fw2), _bt(fb2), _bd(fw3), _bt(fb3))
    return wout.reshape(NE)


def kernel(x, edge_attr, edge_index, params):
    return _run(x, edge_attr, edge_index, params)
